# Initial kernel scaffold; baseline (speedup 1.0000x reference)
import jax
import jax.numpy as jnp
from jax.experimental import pallas as pl


def kernel(inputs, e_feats, rain0, edge_index, W_emb, in_W_src, in_W_edge, in_W_self, in_W_h, in_w_rain, out_W_src, out_W_edge, out_W_self, out_W_h, out_w_rain, out_w_out):
    raise NotImplementedError("write your pallas kernel here")


# speedup vs baseline: 1.5065x; 1.5065x over previous
import functools

import jax
import jax.numpy as jnp
from jax import lax
from jax.experimental import pallas as pl
from jax.experimental.pallas import tpu as pltpu
from jax.experimental.pallas import tpu_sc as plsc

N = 10000
E = 320000
D = 128
ED = 16
L = 4

NC = 2
NS = 16
NW = NC * NS
EPW = E // NW
C = 80
NCHUNK = EPW // C
N_PAD = 10240
RPS = N_PAD // NS
RCHUNK = C
NRC = RPS // RCHUNK


def _edge_pass_body(hs_hbm, em_hbm, src_hbm, dst_hbm, out_hbm,
                    src_c, dst_c, gat_v, msg_v, agg_sh, sem):
    c = lax.axis_index("c")
    s = lax.axis_index("s")
    wid = c * NS + s

    def zbody(i, _):
        r = i // (D // 16)
        col = (i % (D // 16)) * 16
        gat_v[r, pl.ds(col, 16)] = jnp.zeros((16,), jnp.float32)
        return ()

    lax.fori_loop(0, C * (D // 16), zbody, (), unroll=8)
    row0 = s * RPS
    for k in range(NRC):
        pltpu.sync_copy(gat_v, agg_sh.at[pl.ds(row0 + k * RCHUNK, RCHUNK)])
    plsc.subcore_barrier()

    def chunk_body(k, _):
        base = wid * EPW + k * C
        pltpu.sync_copy(src_hbm.at[pl.ds(base, C)], src_c)
        pltpu.sync_copy(dst_hbm.at[pl.ds(base, C)], dst_c)
        pltpu.sync_copy(em_hbm.at[pl.ds(base, C)], msg_v)
        pltpu.async_copy(hs_hbm.at[src_c], gat_v, sem).wait()

        def cbody(i, _):
            r = i // (D // 16)
            col = (i % (D // 16)) * 16
            msg_v[r, pl.ds(col, 16)] = jnp.maximum(
                msg_v[r, pl.ds(col, 16)] + gat_v[r, pl.ds(col, 16)], 0.0)
            return ()

        lax.fori_loop(0, C * (D // 16), cbody, (), unroll=8)
        pltpu.sync_copy(msg_v, agg_sh.at[dst_c], add=True)
        return ()

    lax.fori_loop(0, NCHUNK, chunk_body, ())
    plsc.subcore_barrier()

    for k in range(NRC):
        rows = pl.ds(row0 + k * RCHUNK, RCHUNK)
        pltpu.sync_copy(agg_sh.at[rows], gat_v)
        pltpu.sync_copy(gat_v, out_hbm.at[c, rows])


def _edge_pass(hs, em, src, dst):
    mesh = plsc.VectorSubcoreMesh(core_axis_name="c", subcore_axis_name="s")
    f = pl.kernel(
        _edge_pass_body,
        out_type=jax.ShapeDtypeStruct((NC, N_PAD, D), jnp.float32),
        mesh=mesh,
        scratch_types=[
            pltpu.VMEM((C,), jnp.int32),
            pltpu.VMEM((C,), jnp.int32),
            pltpu.VMEM((C, D), jnp.float32),
            pltpu.VMEM((C, D), jnp.float32),
            pltpu.VMEM_SHARED((N_PAD, D), jnp.float32),
            pltpu.SemaphoreType.DMA,
        ],
    )
    return f(hs, em, src, dst)


BN = 1000


def _embed_body(x_ref, we_ref, ws_ref, h_ref, hs_ref):
    h = x_ref[...] @ we_ref[...]
    h_ref[...] = h
    hs_ref[...] = h @ ws_ref[...]


def _embed(x, W_emb, W_src):
    return pl.pallas_call(
        _embed_body,
        grid=(N // BN,),
        in_specs=[
            pl.BlockSpec((BN, D), lambda i: (i, 0)),
            pl.BlockSpec((D, D), lambda i: (0, 0)),
            pl.BlockSpec((D, D), lambda i: (0, 0)),
        ],
        out_specs=[
            pl.BlockSpec((BN, D), lambda i: (i, 0)),
            pl.BlockSpec((BN, D), lambda i: (i, 0)),
        ],
        out_shape=[
            jax.ShapeDtypeStruct((N, D), jnp.float32),
            jax.ShapeDtypeStruct((N, D), jnp.float32),
        ],
    )(x, W_emb, W_src)


BE = 4000


def _em_body(ef_ref, w_ref, oi_ref, oo_ref):
    res = ef_ref[0] @ w_ref[...]
    oi_ref[0] = res[:, :D]
    oo_ref[0] = res[:, D:]


def _em_all(ef_t, W_cat):
    return pl.pallas_call(
        _em_body,
        grid=(L, E // BE),
        in_specs=[
            pl.BlockSpec((1, BE, ED), lambda l, i: (l, i, 0)),
            pl.BlockSpec((ED, 2 * D), lambda l, i: (0, 0)),
        ],
        out_specs=[
            pl.BlockSpec((1, BE, D), lambda l, i: (l, i, 0)),
            pl.BlockSpec((1, BE, D), lambda l, i: (l, i, 0)),
        ],
        out_shape=[
            jax.ShapeDtypeStruct((L, E, D), jnp.float32),
            jax.ShapeDtypeStruct((L, E, D), jnp.float32),
        ],
    )(ef_t, W_cat)


def _update_body(p_ref, h_ref, r_ref, wself_ref, wh_ref, wrain_ref, wn_ref,
                 wout_ref, hnew_ref, hs_ref, rain_ref):
    agg = p_ref[0] + p_ref[1]
    acc = agg @ wself_ref[...] + h_ref[...] @ wh_ref[...]
    acc = acc + r_ref[...] * wrain_ref[...]
    h_new = jnp.maximum(acc, 0.0)
    hnew_ref[...] = h_new
    hs_ref[...] = h_new @ wn_ref[...]
    rain_ref[...] = h_new @ wout_ref[...]


def _update(p, h, r, W_self, W_h, w_rain, W_next, w_out):
    return pl.pallas_call(
        _update_body,
        grid=(N // BN,),
        in_specs=[
            pl.BlockSpec((NC, BN, D), lambda i: (0, i, 0)),
            pl.BlockSpec((BN, D), lambda i: (i, 0)),
            pl.BlockSpec((BN, 1), lambda i: (i, 0)),
            pl.BlockSpec((D, D), lambda i: (0, 0)),
            pl.BlockSpec((D, D), lambda i: (0, 0)),
            pl.BlockSpec((1, D), lambda i: (0, 0)),
            pl.BlockSpec((D, D), lambda i: (0, 0)),
            pl.BlockSpec((D, 1), lambda i: (0, 0)),
        ],
        out_specs=[
            pl.BlockSpec((BN, D), lambda i: (i, 0)),
            pl.BlockSpec((BN, D), lambda i: (i, 0)),
            pl.BlockSpec((BN, 1), lambda i: (i, 0)),
        ],
        out_shape=[
            jax.ShapeDtypeStruct((N, D), jnp.float32),
            jax.ShapeDtypeStruct((N, D), jnp.float32),
            jax.ShapeDtypeStruct((N, 1), jnp.float32),
        ],
    )(p, h, r, W_self, W_h, w_rain, W_next, w_out)


def kernel(inputs, e_feats, rain0, edge_index, W_emb, in_W_src, in_W_edge,
           in_W_self, in_W_h, in_w_rain, out_W_src, out_W_edge, out_W_self,
           out_W_h, out_w_rain, out_w_out):
    src = edge_index[0]
    dst = edge_index[1]
    ef_t = jnp.transpose(e_feats, (2, 0, 1))
    W_cat = jnp.concatenate([in_W_edge, out_W_edge], axis=1)
    em_in, em_out = _em_all(ef_t, W_cat)

    in_w_rain2 = in_w_rain.reshape(1, D)
    out_w_rain2 = out_w_rain.reshape(1, D)
    w_out2 = out_w_out.reshape(D, 1)

    h, hs = _embed(inputs, W_emb, in_W_src)
    cols = []
    for l in range(L):
        r = rain0[:, l].reshape(N, 1)
        p = _edge_pass(hs, em_in[l], src, dst)
        h, hs, _ = _update(p, h, r, in_W_self, in_W_h, in_w_rain2,
                           out_W_src, w_out2)
        p = _edge_pass(hs, em_out[l], src, dst)
        h, hs, rain_col = _update(p, h, r, out_W_self, out_W_h, out_w_rain2,
                                  in_W_src, w_out2)
        cols.append(rain_col)
    return jnp.concatenate(cols, axis=1)


# speedup vs baseline: 1.8725x; 1.2429x over previous
import functools

import jax
import jax.numpy as jnp
from jax import lax
from jax.experimental import pallas as pl
from jax.experimental.pallas import tpu as pltpu
from jax.experimental.pallas import tpu_sc as plsc

N = 10000
E = 320000
D = 128
ED = 16
L = 4

NC = 2
NS = 16
NW = NC * NS
EPW = E // NW
C = 40
NCHUNK = EPW // C
NRING = 4
PREF = 2
N_PAD = 10240
RPS = N_PAD // NS
RCHUNK = C
NRC = RPS // RCHUNK


def _edge_pass_body(hs_hbm, em_hbm, src_hbm, dst_hbm, out_hbm,
                    src_r, dst_r, gat_r, msg_r, em_sems, gat_sems, sct_sems,
                    agg_sh):
    c = lax.axis_index("c")
    s = lax.axis_index("s")
    wid = c * NS + s

    def zbody(r, _):
        for cc in range(D // 16):
            gat_r[0][r, pl.ds(cc * 16, 16)] = jnp.zeros((16,), jnp.float32)
        return ()

    lax.fori_loop(0, C, zbody, (), unroll=2)
    row0 = s * RPS
    for k in range(NRC):
        pltpu.sync_copy(gat_r[0], agg_sh.at[pl.ds(row0 + k * RCHUNK, RCHUNK)])
    plsc.subcore_barrier()

    def start_chunk(k, p):
        base = wid * EPW + k * C
        pltpu.sync_copy(src_hbm.at[pl.ds(base, C)], src_r[p])
        pltpu.sync_copy(dst_hbm.at[pl.ds(base, C)], dst_r[p])
        pltpu.async_copy(em_hbm.at[pl.ds(base, C)], msg_r[p], em_sems[p])
        pltpu.async_copy(hs_hbm.at[src_r[p]], gat_r[p], gat_sems[p])

    def wait_chunk(p):
        pltpu.make_async_copy(em_hbm.at[pl.ds(0, C)], msg_r[p], em_sems[p]).wait()
        pltpu.make_async_copy(hs_hbm.at[src_r[p]], gat_r[p], gat_sems[p]).wait()

    def compute(p):
        def rbody(r, _):
            for cc in range(D // 16):
                sl = pl.ds(cc * 16, 16)
                msg_r[p][r, sl] = jnp.maximum(
                    msg_r[p][r, sl] + gat_r[p][r, sl], 0.0)
            return ()

        lax.fori_loop(0, C, rbody, (), unroll=2)

    def start_scatter(p):
        pltpu.async_copy(msg_r[p], agg_sh.at[dst_r[p]], sct_sems[p], add=True)

    def wait_scatter(p):
        pltpu.make_async_copy(msg_r[p], agg_sh.at[dst_r[p]], sct_sems[p]).wait()

    start_chunk(0, 0)
    start_chunk(1, 1)

    NGROUP = (NCHUNK + NRING - 1) // NRING

    def group(g, _):
        for p in range(NRING):
            k = g * NRING + p

            @pl.when(k < NCHUNK)
            def _():
                wait_chunk(p)
                compute(p)
                start_scatter(p)
                q = (p + PREF) % NRING

                @pl.when(k >= NRING - PREF)
                def _():
                    wait_scatter(q)

                @pl.when(k + PREF < NCHUNK)
                def _():
                    start_chunk(k + PREF, q)

        return ()

    lax.fori_loop(0, NGROUP, group, ())
    for kk in range(NCHUNK - PREF, NCHUNK):
        wait_scatter(kk % NRING)
    plsc.subcore_barrier()

    for k in range(NRC):
        rows = pl.ds(row0 + k * RCHUNK, RCHUNK)
        pltpu.sync_copy(agg_sh.at[rows], gat_r[0])
        pltpu.sync_copy(gat_r[0], out_hbm.at[c, rows])


def _edge_pass(hs, em, src, dst):
    mesh = plsc.VectorSubcoreMesh(core_axis_name="c", subcore_axis_name="s")
    f = pl.kernel(
        _edge_pass_body,
        out_type=jax.ShapeDtypeStruct((NC, N_PAD, D), jnp.float32),
        mesh=mesh,
        scratch_types=[
            [pltpu.VMEM((C,), jnp.int32) for _ in range(NRING)],
            [pltpu.VMEM((C,), jnp.int32) for _ in range(NRING)],
            [pltpu.VMEM((C, D), jnp.float32) for _ in range(NRING)],
            [pltpu.VMEM((C, D), jnp.float32) for _ in range(NRING)],
            [pltpu.SemaphoreType.DMA for _ in range(NRING)],
            [pltpu.SemaphoreType.DMA for _ in range(NRING)],
            [pltpu.SemaphoreType.DMA for _ in range(NRING)],
            pltpu.VMEM_SHARED((N_PAD, D), jnp.float32),
        ],
    )
    return f(hs, em, src, dst)


BN = 1000


def _embed_body(x_ref, we_ref, ws_ref, h_ref, hs_ref):
    h = x_ref[...] @ we_ref[...]
    h_ref[...] = h
    hs_ref[...] = h @ ws_ref[...]


def _embed(x, W_emb, W_src):
    return pl.pallas_call(
        _embed_body,
        grid=(N // BN,),
        in_specs=[
            pl.BlockSpec((BN, D), lambda i: (i, 0)),
            pl.BlockSpec((D, D), lambda i: (0, 0)),
            pl.BlockSpec((D, D), lambda i: (0, 0)),
        ],
        out_specs=[
            pl.BlockSpec((BN, D), lambda i: (i, 0)),
            pl.BlockSpec((BN, D), lambda i: (i, 0)),
        ],
        out_shape=[
            jax.ShapeDtypeStruct((N, D), jnp.float32),
            jax.ShapeDtypeStruct((N, D), jnp.float32),
        ],
    )(x, W_emb, W_src)


BE = 4000


def _em_body(ef_ref, w_ref, oi_ref, oo_ref):
    res = ef_ref[0] @ w_ref[...]
    oi_ref[0] = res[:, :D]
    oo_ref[0] = res[:, D:]


def _em_all(ef_t, W_cat):
    return pl.pallas_call(
        _em_body,
        grid=(L, E // BE),
        in_specs=[
            pl.BlockSpec((1, BE, ED), lambda l, i: (l, i, 0)),
            pl.BlockSpec((ED, 2 * D), lambda l, i: (0, 0)),
        ],
        out_specs=[
            pl.BlockSpec((1, BE, D), lambda l, i: (l, i, 0)),
            pl.BlockSpec((1, BE, D), lambda l, i: (l, i, 0)),
        ],
        out_shape=[
            jax.ShapeDtypeStruct((L, E, D), jnp.float32),
            jax.ShapeDtypeStruct((L, E, D), jnp.float32),
        ],
    )(ef_t, W_cat)


def _update_body(p_ref, h_ref, r_ref, wself_ref, wh_ref, wrain_ref, wn_ref,
                 wout_ref, hnew_ref, hs_ref, rain_ref):
    agg = p_ref[0] + p_ref[1]
    acc = agg @ wself_ref[...] + h_ref[...] @ wh_ref[...]
    acc = acc + r_ref[...] * wrain_ref[...]
    h_new = jnp.maximum(acc, 0.0)
    hnew_ref[...] = h_new
    hs_ref[...] = h_new @ wn_ref[...]
    rain_ref[...] = h_new @ wout_ref[...]


def _update(p, h, r, W_self, W_h, w_rain, W_next, w_out):
    return pl.pallas_call(
        _update_body,
        grid=(N // BN,),
        in_specs=[
            pl.BlockSpec((NC, BN, D), lambda i: (0, i, 0)),
            pl.BlockSpec((BN, D), lambda i: (i, 0)),
            pl.BlockSpec((BN, 1), lambda i: (i, 0)),
            pl.BlockSpec((D, D), lambda i: (0, 0)),
            pl.BlockSpec((D, D), lambda i: (0, 0)),
            pl.BlockSpec((1, D), lambda i: (0, 0)),
            pl.BlockSpec((D, D), lambda i: (0, 0)),
            pl.BlockSpec((D, 1), lambda i: (0, 0)),
        ],
        out_specs=[
            pl.BlockSpec((BN, D), lambda i: (i, 0)),
            pl.BlockSpec((BN, D), lambda i: (i, 0)),
            pl.BlockSpec((BN, 1), lambda i: (i, 0)),
        ],
        out_shape=[
            jax.ShapeDtypeStruct((N, D), jnp.float32),
            jax.ShapeDtypeStruct((N, D), jnp.float32),
            jax.ShapeDtypeStruct((N, 1), jnp.float32),
        ],
    )(p, h, r, W_self, W_h, w_rain, W_next, w_out)


def kernel(inputs, e_feats, rain0, edge_index, W_emb, in_W_src, in_W_edge,
           in_W_self, in_W_h, in_w_rain, out_W_src, out_W_edge, out_W_self,
           out_W_h, out_w_rain, out_w_out):
    src = edge_index[0]
    dst = edge_index[1]
    ef_t = jnp.transpose(e_feats, (2, 0, 1))
    W_cat = jnp.concatenate([in_W_edge, out_W_edge], axis=1)
    em_in, em_out = _em_all(ef_t, W_cat)

    in_w_rain2 = in_w_rain.reshape(1, D)
    out_w_rain2 = out_w_rain.reshape(1, D)
    w_out2 = out_w_out.reshape(D, 1)

    h, hs = _embed(inputs, W_emb, in_W_src)
    cols = []
    for l in range(L):
        r = rain0[:, l].reshape(N, 1)
        p = _edge_pass(hs, em_in[l], src, dst)
        h, hs, _ = _update(p, h, r, in_W_self, in_W_h, in_w_rain2,
                           out_W_src, w_out2)
        p = _edge_pass(hs, em_out[l], src, dst)
        h, hs, rain_col = _update(p, h, r, out_W_self, out_W_h, out_w_rain2,
                                  in_W_src, w_out2)
        cols.append(rain_col)
    return jnp.concatenate(cols, axis=1)


# speedup vs baseline: 2.4668x; 1.3174x over previous
import functools

import jax
import jax.numpy as jnp
from jax import lax
from jax.experimental import pallas as pl
from jax.experimental.pallas import tpu as pltpu
from jax.experimental.pallas import tpu_sc as plsc

N = 10000
E = 320000
D = 128
ED = 16
L = 4

NC = 2
NS = 16
NW = NC * NS
EPW = E // NW
C = 40
NCHUNK = EPW // C
NRING = 4
PREF = 2
NIRING = 8
IPREF = 4
N_PAD = 10240
RPS = N_PAD // NS
RCHUNK = C
NRC = RPS // RCHUNK


def _edge_pass_body(hs_hbm, em_hbm, src_hbm, dst_hbm, out_hbm,
                    src_r, dst_r, gat_r, msg_r, src_sems, dst_sems,
                    em_sems, gat_sems, sct_sems, agg_sh):
    c = lax.axis_index("c")
    s = lax.axis_index("s")
    wid = c * NS + s

    def zbody(r, _):
        for cc in range(D // 16):
            gat_r[0][r, pl.ds(cc * 16, 16)] = jnp.zeros((16,), jnp.float32)
        return ()

    lax.fori_loop(0, C, zbody, (), unroll=2)
    row0 = s * RPS
    for k in range(NRC):
        pltpu.sync_copy(gat_r[0], agg_sh.at[pl.ds(row0 + k * RCHUNK, RCHUNK)])
    plsc.subcore_barrier()

    def start_idx(k, j):
        base = wid * EPW + k * C
        pltpu.async_copy(src_hbm.at[pl.ds(base, C)], src_r[j], src_sems[j])
        pltpu.async_copy(dst_hbm.at[pl.ds(base, C)], dst_r[j], dst_sems[j])

    def wait_src(j):
        pltpu.make_async_copy(src_hbm.at[pl.ds(0, C)], src_r[j], src_sems[j]).wait()

    def wait_dst(j):
        pltpu.make_async_copy(dst_hbm.at[pl.ds(0, C)], dst_r[j], dst_sems[j]).wait()

    def start_data(k, p, j):
        base = wid * EPW + k * C
        pltpu.async_copy(em_hbm.at[pl.ds(base, C)], msg_r[p], em_sems[p])
        pltpu.async_copy(hs_hbm.at[src_r[j]], gat_r[p], gat_sems[p])

    def wait_data(p, j):
        pltpu.make_async_copy(em_hbm.at[pl.ds(0, C)], msg_r[p], em_sems[p]).wait()
        pltpu.make_async_copy(hs_hbm.at[src_r[j]], gat_r[p], gat_sems[p]).wait()

    def compute(p):
        def rbody(r, _):
            for cc in range(D // 16):
                sl = pl.ds(cc * 16, 16)
                msg_r[p][r, sl] = jnp.maximum(
                    msg_r[p][r, sl] + gat_r[p][r, sl], 0.0)
            return ()

        lax.fori_loop(0, C, rbody, (), unroll=2)

    def start_scatter(p, j):
        pltpu.async_copy(msg_r[p], agg_sh.at[dst_r[j]], sct_sems[p], add=True)

    def wait_scatter(p, j):
        pltpu.make_async_copy(msg_r[p], agg_sh.at[dst_r[j]], sct_sems[p]).wait()

    for kk in range(IPREF):
        start_idx(kk, kk % NIRING)
    for kk in range(PREF):
        wait_src(kk % NIRING)
        start_data(kk, kk % NRING, kk % NIRING)

    NGROUP = (NCHUNK + NIRING - 1) // NIRING

    def group(g, _):
        for j in range(NIRING):
            k = g * NIRING + j
            p = j % NRING

            @pl.when(k < NCHUNK)
            def _():
                wait_data(p, j)
                wait_dst(j)
                compute(p)
                start_scatter(p, j)
                q = (p + PREF) % NRING
                jq = (j + PREF) % NIRING

                @pl.when(k >= NRING - PREF)
                def _():
                    wait_scatter(q, (j + NIRING - PREF) % NIRING)

                @pl.when(k + PREF < NCHUNK)
                def _():
                    wait_src(jq)
                    start_data(k + PREF, q, jq)

                @pl.when(k + IPREF < NCHUNK)
                def _():
                    start_idx(k + IPREF, (j + IPREF) % NIRING)

        return ()

    lax.fori_loop(0, NGROUP, group, ())
    for kk in range(NCHUNK - PREF, NCHUNK):
        wait_scatter(kk % NRING, kk % NIRING)
    plsc.subcore_barrier()

    for k in range(NRC):
        rows = pl.ds(row0 + k * RCHUNK, RCHUNK)
        pltpu.sync_copy(agg_sh.at[rows], gat_r[0])
        pltpu.sync_copy(gat_r[0], out_hbm.at[c, rows])


def _edge_pass(hs, em, src, dst):
    mesh = plsc.VectorSubcoreMesh(core_axis_name="c", subcore_axis_name="s")
    f = pl.kernel(
        _edge_pass_body,
        out_type=jax.ShapeDtypeStruct((NC, N_PAD, D), jnp.float32),
        mesh=mesh,
        scratch_types=[
            [pltpu.VMEM((C,), jnp.int32) for _ in range(NIRING)],
            [pltpu.VMEM((C,), jnp.int32) for _ in range(NIRING)],
            [pltpu.VMEM((C, D), jnp.float32) for _ in range(NRING)],
            [pltpu.VMEM((C, D), jnp.float32) for _ in range(NRING)],
            [pltpu.SemaphoreType.DMA for _ in range(NIRING)],
            [pltpu.SemaphoreType.DMA for _ in range(NIRING)],
            [pltpu.SemaphoreType.DMA for _ in range(NRING)],
            [pltpu.SemaphoreType.DMA for _ in range(NRING)],
            [pltpu.SemaphoreType.DMA for _ in range(NRING)],
            pltpu.VMEM_SHARED((N_PAD, D), jnp.float32),
        ],
    )
    return f(hs, em, src, dst)


BN = 1000


def _embed_body(x_ref, we_ref, ws_ref, h_ref, hs_ref):
    h = x_ref[...] @ we_ref[...]
    h_ref[...] = h
    hs_ref[...] = h @ ws_ref[...]


def _embed(x, W_emb, W_src):
    return pl.pallas_call(
        _embed_body,
        grid=(N // BN,),
        in_specs=[
            pl.BlockSpec((BN, D), lambda i: (i, 0)),
            pl.BlockSpec((D, D), lambda i: (0, 0)),
            pl.BlockSpec((D, D), lambda i: (0, 0)),
        ],
        out_specs=[
            pl.BlockSpec((BN, D), lambda i: (i, 0)),
            pl.BlockSpec((BN, D), lambda i: (i, 0)),
        ],
        out_shape=[
            jax.ShapeDtypeStruct((N, D), jnp.float32),
            jax.ShapeDtypeStruct((N, D), jnp.float32),
        ],
    )(x, W_emb, W_src)


BE = 4000


def _em_body(ef_ref, w_ref, oi_ref, oo_ref):
    res = ef_ref[0] @ w_ref[...]
    oi_ref[0] = res[:, :D]
    oo_ref[0] = res[:, D:]


def _em_all(ef_t, W_cat):
    return pl.pallas_call(
        _em_body,
        grid=(L, E // BE),
        in_specs=[
            pl.BlockSpec((1, BE, ED), lambda l, i: (l, i, 0)),
            pl.BlockSpec((ED, 2 * D), lambda l, i: (0, 0)),
        ],
        out_specs=[
            pl.BlockSpec((1, BE, D), lambda l, i: (l, i, 0)),
            pl.BlockSpec((1, BE, D), lambda l, i: (l, i, 0)),
        ],
        out_shape=[
            jax.ShapeDtypeStruct((L, E, D), jnp.float32),
            jax.ShapeDtypeStruct((L, E, D), jnp.float32),
        ],
    )(ef_t, W_cat)


def _update_body(p_ref, h_ref, r_ref, wself_ref, wh_ref, wrain_ref, wn_ref,
                 wout_ref, hnew_ref, hs_ref, rain_ref):
    agg = p_ref[0] + p_ref[1]
    acc = agg @ wself_ref[...] + h_ref[...] @ wh_ref[...]
    acc = acc + r_ref[...] * wrain_ref[...]
    h_new = jnp.maximum(acc, 0.0)
    hnew_ref[...] = h_new
    hs_ref[...] = h_new @ wn_ref[...]
    rain_ref[...] = h_new @ wout_ref[...]


def _update(p, h, r, W_self, W_h, w_rain, W_next, w_out):
    return pl.pallas_call(
        _update_body,
        grid=(N // BN,),
        in_specs=[
            pl.BlockSpec((NC, BN, D), lambda i: (0, i, 0)),
            pl.BlockSpec((BN, D), lambda i: (i, 0)),
            pl.BlockSpec((BN, 1), lambda i: (i, 0)),
            pl.BlockSpec((D, D), lambda i: (0, 0)),
            pl.BlockSpec((D, D), lambda i: (0, 0)),
            pl.BlockSpec((1, D), lambda i: (0, 0)),
            pl.BlockSpec((D, D), lambda i: (0, 0)),
            pl.BlockSpec((D, 1), lambda i: (0, 0)),
        ],
        out_specs=[
            pl.BlockSpec((BN, D), lambda i: (i, 0)),
            pl.BlockSpec((BN, D), lambda i: (i, 0)),
            pl.BlockSpec((BN, 1), lambda i: (i, 0)),
        ],
        out_shape=[
            jax.ShapeDtypeStruct((N, D), jnp.float32),
            jax.ShapeDtypeStruct((N, D), jnp.float32),
            jax.ShapeDtypeStruct((N, 1), jnp.float32),
        ],
    )(p, h, r, W_self, W_h, w_rain, W_next, w_out)


def kernel(inputs, e_feats, rain0, edge_index, W_emb, in_W_src, in_W_edge,
           in_W_self, in_W_h, in_w_rain, out_W_src, out_W_edge, out_W_self,
           out_W_h, out_w_rain, out_w_out):
    src = edge_index[0]
    dst = edge_index[1]
    ef_t = jnp.transpose(e_feats, (2, 0, 1))
    W_cat = jnp.concatenate([in_W_edge, out_W_edge], axis=1)
    em_in, em_out = _em_all(ef_t, W_cat)

    in_w_rain2 = in_w_rain.reshape(1, D)
    out_w_rain2 = out_w_rain.reshape(1, D)
    w_out2 = out_w_out.reshape(D, 1)

    h, hs = _embed(inputs, W_emb, in_W_src)
    cols = []
    for l in range(L):
        r = rain0[:, l].reshape(N, 1)
        p = _edge_pass(hs, em_in[l], src, dst)
        h, hs, _ = _update(p, h, r, in_W_self, in_W_h, in_w_rain2,
                           out_W_src, w_out2)
        p = _edge_pass(hs, em_out[l], src, dst)
        h, hs, rain_col = _update(p, h, r, out_W_self, out_W_h, out_w_rain2,
                                  in_W_src, w_out2)
        cols.append(rain_col)
    return jnp.concatenate(cols, axis=1)
